# baseline (speedup 1.0000x reference)
import functools

import jax
import jax.numpy as jnp
from jax import lax
from jax.experimental import pallas as pl
from jax.experimental.pallas import tpu as pltpu
from jax.experimental.pallas import tpu_sc as plsc

N = 10000
D = 128
E = 320000
L = 3

NC = 2
NS = 16
NW = NC * NS

CHUNK = 128
CPW = 80
EPW = CHUNK * CPW
E_PAD = NW * EPW
N_PAD = 10240
ROWS_PER_TILE = N_PAD // NS
DUMMY_DST = N_PAD - 8
NBUF = 2
GLA = 1
NI = 3


def _sc_agg_body(h_hbm, ei_hbm, out_hbm, idx, rows, agg_sh,
                 sem_i, sem_g, sem_s):
    c = lax.axis_index("c")
    s = lax.axis_index("s")
    wid = s * NC + c

    def _zrow(i, _):
        for j in range(D // 16):
            rows[0, i, pl.ds(j * 16, 16)] = jnp.zeros((16,), jnp.float32)
        return _

    lax.fori_loop(0, CHUNK, _zrow, None)
    for t in range(ROWS_PER_TILE // CHUNK):
        pltpu.sync_copy(rows.at[0],
                        agg_sh.at[pl.ds(s * ROWS_PER_TILE + t * CHUNK, CHUNK)])
    plsc.subcore_barrier()

    def _start_i(j):
        sl = lax.rem(j, NI)
        pltpu.async_copy(ei_hbm.at[wid, j], idx.at[sl], sem_i.at[sl])

    def _wait_i(j):
        sl = lax.rem(j, NI)
        pltpu.make_async_copy(ei_hbm.at[wid, j], idx.at[sl],
                              sem_i.at[sl]).wait()

    def _start_g(j):
        sl = lax.rem(j, NI)
        b = lax.rem(j, NBUF)
        pltpu.async_copy(h_hbm.at[idx.at[sl, 0]], rows.at[b], sem_g.at[b])

    def _wait_g(j):
        sl = lax.rem(j, NI)
        b = lax.rem(j, NBUF)
        pltpu.make_async_copy(h_hbm.at[idx.at[sl, 0]], rows.at[b],
                              sem_g.at[b]).wait()

    def _start_s(j):
        sl = lax.rem(j, NI)
        b = lax.rem(j, NBUF)
        pltpu.async_copy(rows.at[b], agg_sh.at[idx.at[sl, 1]], sem_s.at[b],
                         add=True)

    def _wait_s(j):
        sl = lax.rem(j, NI)
        b = lax.rem(j, NBUF)
        pltpu.make_async_copy(rows.at[b], agg_sh.at[idx.at[sl, 1]],
                              sem_s.at[b]).wait()

    for j in range(GLA + 1):
        _start_i(j)
    for j in range(GLA):
        _wait_i(j)
        _start_g(j)

    def _iter(i, _):
        _wait_g(i)
        _start_s(i)
        j2 = i + GLA + 1

        @pl.when(j2 < CPW)
        def _():
            _start_i(j2)
        j = i + GLA

        @pl.when(j < CPW)
        def _():
            @pl.when(j >= NBUF)
            def _():
                _wait_s(j - NBUF)
            _wait_i(j)
            _start_g(j)
        return _

    lax.fori_loop(0, CPW, _iter, None)
    for j in range(CPW - NBUF, CPW):
        _wait_s(j)
    plsc.subcore_barrier()

    for t in range(ROWS_PER_TILE // CHUNK):
        r = s * ROWS_PER_TILE + t * CHUNK
        pltpu.sync_copy(agg_sh.at[pl.ds(r, CHUNK)], out_hbm.at[c, pl.ds(r, CHUNK)])


_sc_agg = pl.kernel(
    _sc_agg_body,
    out_type=jax.ShapeDtypeStruct((NC, N_PAD, D), jnp.float32),
    mesh=plsc.VectorSubcoreMesh(core_axis_name="c", subcore_axis_name="s"),
    scratch_types=[
        pltpu.VMEM((NI, 2, CHUNK), jnp.int32),
        pltpu.VMEM((NBUF, CHUNK, D), jnp.float32),
        pltpu.VMEM_SHARED((N_PAD, D), jnp.float32),
        pltpu.SemaphoreType.DMA((NI,)),
        pltpu.SemaphoreType.DMA((NBUF,)),
        pltpu.SemaphoreType.DMA((NBUF,)),
    ],
)


BLK = 400
GRID = N // BLK


def _dense_body(h_ref, a0_ref, a1_ref, w1_ref, b1_ref, w2_ref, b2_ref,
                z_ref, s_ref, ss_ref):
    i = pl.program_id(0)
    zin = h_ref[...] + a0_ref[0] + a1_ref[0]
    t = jnp.dot(zin, w1_ref[...], preferred_element_type=jnp.float32)
    t = jnp.maximum(t + b1_ref[...], 0.0)
    z2 = jnp.dot(t, w2_ref[...], preferred_element_type=jnp.float32)
    z2 = jnp.maximum(z2 + b2_ref[...], 0.0)
    z_ref[...] = z2
    ps = jnp.sum(z2.reshape(BLK // 8, 8, D), axis=0)
    pss = jnp.sum((z2 * z2).reshape(BLK // 8, 8, D), axis=0)

    @pl.when(i == 0)
    def _init():
        s_ref[...] = ps
        ss_ref[...] = pss

    @pl.when(i > 0)
    def _acc():
        s_ref[...] += ps
        ss_ref[...] += pss


_dense = pl.pallas_call(
    _dense_body,
    grid=(GRID,),
    in_specs=[
        pl.BlockSpec((BLK, D), lambda i: (i, 0)),
        pl.BlockSpec((1, BLK, D), lambda i: (0, i, 0)),
        pl.BlockSpec((1, BLK, D), lambda i: (1, i, 0)),
        pl.BlockSpec((D, D), lambda i: (0, 0)),
        pl.BlockSpec((1, D), lambda i: (0, 0)),
        pl.BlockSpec((D, D), lambda i: (0, 0)),
        pl.BlockSpec((1, D), lambda i: (0, 0)),
    ],
    out_specs=[
        pl.BlockSpec((BLK, D), lambda i: (i, 0)),
        pl.BlockSpec((8, D), lambda i: (0, 0)),
        pl.BlockSpec((8, D), lambda i: (0, 0)),
    ],
    out_shape=[
        jax.ShapeDtypeStruct((N, D), jnp.float32),
        jax.ShapeDtypeStruct((8, D), jnp.float32),
        jax.ShapeDtypeStruct((8, D), jnp.float32),
    ],
)


def _norm_body(z_ref, s_ref, ss_ref, g_ref, b_ref, o_ref):
    mean = jnp.sum(s_ref[...], axis=0, keepdims=True) * (1.0 / N)
    msq = jnp.sum(ss_ref[...], axis=0, keepdims=True) * (1.0 / N)
    var = msq - mean * mean
    inv = lax.rsqrt(var + 1e-5)
    scale = g_ref[...] * inv
    shift = b_ref[...] - mean * scale
    o_ref[...] = z_ref[...] * scale + shift


_norm = pl.pallas_call(
    _norm_body,
    grid=(GRID,),
    in_specs=[
        pl.BlockSpec((BLK, D), lambda i: (i, 0)),
        pl.BlockSpec((8, D), lambda i: (0, 0)),
        pl.BlockSpec((8, D), lambda i: (0, 0)),
        pl.BlockSpec((1, D), lambda i: (0, 0)),
        pl.BlockSpec((1, D), lambda i: (0, 0)),
    ],
    out_specs=pl.BlockSpec((BLK, D), lambda i: (i, 0)),
    out_shape=jax.ShapeDtypeStruct((N, D), jnp.float32),
)


def kernel(x, edge_index,
           W1_0, b1_0, W2_0, b2_0, gamma_0, beta_0,
           W1_1, b1_1, W2_1, b2_1, gamma_1, beta_1,
           W1_2, b1_2, W2_2, b2_2, gamma_2, beta_2):
    params = [
        (W1_0, b1_0, W2_0, b2_0, gamma_0, beta_0),
        (W1_1, b1_1, W2_1, b2_1, gamma_1, beta_1),
        (W1_2, b1_2, W2_2, b2_2, gamma_2, beta_2),
    ]
    npad = E_PAD - E
    pad = jnp.stack([
        jnp.arange(npad, dtype=jnp.int32) % N,
        N + (jnp.arange(npad, dtype=jnp.int32) % (N_PAD - N)),
    ])
    ei = jnp.concatenate([edge_index, pad], axis=1)
    ei = ei.reshape(2, NW, CPW, CHUNK).transpose(1, 2, 0, 3)

    h = x
    for (W1, b1, W2, b2, g, be) in params:
        agg = _sc_agg(h, ei)
        z, s, ss = _dense(h, agg, agg, W1, b1.reshape(1, D), W2, b2.reshape(1, D))
        h = _norm(z, s, ss, g.reshape(1, D), be.reshape(1, D))
    return h


# speedup vs baseline: 1.1651x; 1.1651x over previous
import functools

import jax
import jax.numpy as jnp
from jax import lax
from jax.experimental import pallas as pl
from jax.experimental.pallas import tpu as pltpu
from jax.experimental.pallas import tpu_sc as plsc

N = 10000
D = 128
E = 320000
L = 3

NC = 2
NS = 16
NW = NC * NS

CHUNK = 64
CPW = 160
EPW = CHUNK * CPW
E_PAD = NW * EPW
N_PAD = 10240
ROWS_PER_TILE = N_PAD // NS
DUMMY_DST = N_PAD - 8
NBUF = 5
GLA = 3
NI = 5


def _sc_agg_body(h_hbm, ei_hbm, out_hbm, idx, rows, agg_sh,
                 sem_i, sem_g, sem_s):
    c = lax.axis_index("c")
    s = lax.axis_index("s")
    wid = s * NC + c

    def _zrow(i, _):
        for j in range(D // 16):
            rows[0, i, pl.ds(j * 16, 16)] = jnp.zeros((16,), jnp.float32)
        return _

    lax.fori_loop(0, CHUNK, _zrow, None)
    for t in range(ROWS_PER_TILE // CHUNK):
        pltpu.sync_copy(rows.at[0],
                        agg_sh.at[pl.ds(s * ROWS_PER_TILE + t * CHUNK, CHUNK)])
    plsc.subcore_barrier()

    def _start_i(j):
        sl = lax.rem(j, NI)
        pltpu.async_copy(ei_hbm.at[wid, j], idx.at[sl], sem_i.at[sl])

    def _wait_i(j):
        sl = lax.rem(j, NI)
        pltpu.make_async_copy(ei_hbm.at[wid, j], idx.at[sl],
                              sem_i.at[sl]).wait()

    def _start_g(j):
        sl = lax.rem(j, NI)
        b = lax.rem(j, NBUF)
        pltpu.async_copy(h_hbm.at[idx.at[sl, 0]], rows.at[b], sem_g.at[b])

    def _wait_g(j):
        sl = lax.rem(j, NI)
        b = lax.rem(j, NBUF)
        pltpu.make_async_copy(h_hbm.at[idx.at[sl, 0]], rows.at[b],
                              sem_g.at[b]).wait()

    def _start_s(j):
        sl = lax.rem(j, NI)
        b = lax.rem(j, NBUF)
        pltpu.async_copy(rows.at[b], agg_sh.at[idx.at[sl, 1]], sem_s.at[b],
                         add=True)

    def _wait_s(j):
        sl = lax.rem(j, NI)
        b = lax.rem(j, NBUF)
        pltpu.make_async_copy(rows.at[b], agg_sh.at[idx.at[sl, 1]],
                              sem_s.at[b]).wait()

    for j in range(GLA + 1):
        _start_i(j)
    for j in range(GLA):
        _wait_i(j)
        _start_g(j)

    def _iter(i, _):
        _wait_g(i)
        _start_s(i)
        j2 = i + GLA + 1

        @pl.when(j2 < CPW)
        def _():
            _start_i(j2)
        j = i + GLA

        @pl.when(j < CPW)
        def _():
            @pl.when(j >= NBUF)
            def _():
                _wait_s(j - NBUF)
            _wait_i(j)
            _start_g(j)
        return _

    lax.fori_loop(0, CPW, _iter, None)
    for j in range(CPW - NBUF, CPW):
        _wait_s(j)
    plsc.subcore_barrier()

    for t in range(ROWS_PER_TILE // CHUNK):
        r = s * ROWS_PER_TILE + t * CHUNK
        pltpu.sync_copy(agg_sh.at[pl.ds(r, CHUNK)], out_hbm.at[c, pl.ds(r, CHUNK)])


_sc_agg = pl.kernel(
    _sc_agg_body,
    out_type=jax.ShapeDtypeStruct((NC, N_PAD, D), jnp.float32),
    mesh=plsc.VectorSubcoreMesh(core_axis_name="c", subcore_axis_name="s"),
    scratch_types=[
        pltpu.VMEM((NI, 2, CHUNK), jnp.int32),
        pltpu.VMEM((NBUF, CHUNK, D), jnp.float32),
        pltpu.VMEM_SHARED((N_PAD, D), jnp.float32),
        pltpu.SemaphoreType.DMA((NI,)),
        pltpu.SemaphoreType.DMA((NBUF,)),
        pltpu.SemaphoreType.DMA((NBUF,)),
    ],
)


BLK = 400
GRID = N // BLK


def _dense_body(h_ref, a0_ref, a1_ref, w1_ref, b1_ref, w2_ref, b2_ref,
                z_ref, s_ref, ss_ref):
    i = pl.program_id(0)
    zin = h_ref[...] + a0_ref[0] + a1_ref[0]
    t = jnp.dot(zin, w1_ref[...], preferred_element_type=jnp.float32)
    t = jnp.maximum(t + b1_ref[...], 0.0)
    z2 = jnp.dot(t, w2_ref[...], preferred_element_type=jnp.float32)
    z2 = jnp.maximum(z2 + b2_ref[...], 0.0)
    z_ref[...] = z2
    ps = jnp.sum(z2.reshape(BLK // 8, 8, D), axis=0)
    pss = jnp.sum((z2 * z2).reshape(BLK // 8, 8, D), axis=0)

    @pl.when(i == 0)
    def _init():
        s_ref[...] = ps
        ss_ref[...] = pss

    @pl.when(i > 0)
    def _acc():
        s_ref[...] += ps
        ss_ref[...] += pss


_dense = pl.pallas_call(
    _dense_body,
    grid=(GRID,),
    in_specs=[
        pl.BlockSpec((BLK, D), lambda i: (i, 0)),
        pl.BlockSpec((1, BLK, D), lambda i: (0, i, 0)),
        pl.BlockSpec((1, BLK, D), lambda i: (1, i, 0)),
        pl.BlockSpec((D, D), lambda i: (0, 0)),
        pl.BlockSpec((1, D), lambda i: (0, 0)),
        pl.BlockSpec((D, D), lambda i: (0, 0)),
        pl.BlockSpec((1, D), lambda i: (0, 0)),
    ],
    out_specs=[
        pl.BlockSpec((BLK, D), lambda i: (i, 0)),
        pl.BlockSpec((8, D), lambda i: (0, 0)),
        pl.BlockSpec((8, D), lambda i: (0, 0)),
    ],
    out_shape=[
        jax.ShapeDtypeStruct((N, D), jnp.float32),
        jax.ShapeDtypeStruct((8, D), jnp.float32),
        jax.ShapeDtypeStruct((8, D), jnp.float32),
    ],
)


def _norm_body(z_ref, s_ref, ss_ref, g_ref, b_ref, o_ref):
    mean = jnp.sum(s_ref[...], axis=0, keepdims=True) * (1.0 / N)
    msq = jnp.sum(ss_ref[...], axis=0, keepdims=True) * (1.0 / N)
    var = msq - mean * mean
    inv = lax.rsqrt(var + 1e-5)
    scale = g_ref[...] * inv
    shift = b_ref[...] - mean * scale
    o_ref[...] = z_ref[...] * scale + shift


_norm = pl.pallas_call(
    _norm_body,
    grid=(GRID,),
    in_specs=[
        pl.BlockSpec((BLK, D), lambda i: (i, 0)),
        pl.BlockSpec((8, D), lambda i: (0, 0)),
        pl.BlockSpec((8, D), lambda i: (0, 0)),
        pl.BlockSpec((1, D), lambda i: (0, 0)),
        pl.BlockSpec((1, D), lambda i: (0, 0)),
    ],
    out_specs=pl.BlockSpec((BLK, D), lambda i: (i, 0)),
    out_shape=jax.ShapeDtypeStruct((N, D), jnp.float32),
)


def kernel(x, edge_index,
           W1_0, b1_0, W2_0, b2_0, gamma_0, beta_0,
           W1_1, b1_1, W2_1, b2_1, gamma_1, beta_1,
           W1_2, b1_2, W2_2, b2_2, gamma_2, beta_2):
    params = [
        (W1_0, b1_0, W2_0, b2_0, gamma_0, beta_0),
        (W1_1, b1_1, W2_1, b2_1, gamma_1, beta_1),
        (W1_2, b1_2, W2_2, b2_2, gamma_2, beta_2),
    ]
    npad = E_PAD - E
    pad = jnp.stack([
        jnp.arange(npad, dtype=jnp.int32) % N,
        N + (jnp.arange(npad, dtype=jnp.int32) % (N_PAD - N)),
    ])
    ei = jnp.concatenate([edge_index, pad], axis=1)
    ei = ei.reshape(2, NW, CPW, CHUNK).transpose(1, 2, 0, 3)

    h = x
    for (W1, b1, W2, b2, g, be) in params:
        agg = _sc_agg(h, ei)
        z, s, ss = _dense(h, agg, agg, W1, b1.reshape(1, D), W2, b2.reshape(1, D))
        h = _norm(z, s, ss, g.reshape(1, D), be.reshape(1, D))
    return h


# speedup vs baseline: 1.2305x; 1.0562x over previous
import functools

import jax
import jax.numpy as jnp
from jax import lax
from jax.experimental import pallas as pl
from jax.experimental.pallas import tpu as pltpu
from jax.experimental.pallas import tpu_sc as plsc

N = 10000
D = 128
E = 320000
L = 3

NC = 2
NS = 16
NW = NC * NS

CHUNK = 64
CPW = 160
EPW = CHUNK * CPW
E_PAD = NW * EPW
N_PAD = 10240
ROWS_PER_TILE = N_PAD // NS
DUMMY_DST = N_PAD - 8
NBUF = 5
GLA = 3
NI = 5


def _sc_agg_body(h_hbm, ei_hbm, out_hbm, idx, rows, agg_sh,
                 sem_i, sem_g, sem_s):
    c = lax.axis_index("c")
    s = lax.axis_index("s")
    wid = s * NC + c

    def _zrow(i, _):
        for j in range(D // 16):
            rows[0, i, pl.ds(j * 16, 16)] = jnp.zeros((16,), jnp.float32)
        return _

    lax.fori_loop(0, CHUNK, _zrow, None)
    for t in range(ROWS_PER_TILE // CHUNK):
        pltpu.sync_copy(rows.at[0],
                        agg_sh.at[pl.ds(s * ROWS_PER_TILE + t * CHUNK, CHUNK)])
    plsc.subcore_barrier()

    def _start_i(j):
        sl = lax.rem(j, NI)
        pltpu.async_copy(ei_hbm.at[wid, j], idx.at[sl], sem_i.at[sl])

    def _wait_i(j):
        sl = lax.rem(j, NI)
        pltpu.make_async_copy(ei_hbm.at[wid, j], idx.at[sl],
                              sem_i.at[sl]).wait()

    def _start_g(j):
        sl = lax.rem(j, NI)
        b = lax.rem(j, NBUF)
        pltpu.async_copy(h_hbm.at[idx.at[sl, 0]], rows.at[b], sem_g.at[b])

    def _wait_g(j):
        sl = lax.rem(j, NI)
        b = lax.rem(j, NBUF)
        pltpu.make_async_copy(h_hbm.at[idx.at[sl, 0]], rows.at[b],
                              sem_g.at[b]).wait()

    def _start_s(j):
        sl = lax.rem(j, NI)
        b = lax.rem(j, NBUF)
        pltpu.async_copy(rows.at[b], agg_sh.at[idx.at[sl, 1]], sem_s.at[b],
                         add=True)

    def _wait_s(j):
        sl = lax.rem(j, NI)
        b = lax.rem(j, NBUF)
        pltpu.make_async_copy(rows.at[b], agg_sh.at[idx.at[sl, 1]],
                              sem_s.at[b]).wait()

    for j in range(GLA + 1):
        _start_i(j)
    for j in range(GLA):
        _wait_i(j)
        _start_g(j)

    def _iter(i, _):
        _wait_g(i)
        _start_s(i)
        j2 = i + GLA + 1

        @pl.when(j2 < CPW)
        def _():
            _start_i(j2)
        j = i + GLA

        @pl.when(j < CPW)
        def _():
            @pl.when(j >= NBUF)
            def _():
                _wait_s(j - NBUF)
            _wait_i(j)
            _start_g(j)
        return _

    lax.fori_loop(0, CPW, _iter, None)
    for j in range(CPW - NBUF, CPW):
        _wait_s(j)
    plsc.subcore_barrier()

    for t in range(ROWS_PER_TILE // CHUNK):
        r = s * ROWS_PER_TILE + t * CHUNK
        pltpu.sync_copy(agg_sh.at[pl.ds(r, CHUNK)], out_hbm.at[c, pl.ds(r, CHUNK)])


_sc_agg = pl.kernel(
    _sc_agg_body,
    out_type=jax.ShapeDtypeStruct((NC, N_PAD, D), jnp.float32),
    mesh=plsc.VectorSubcoreMesh(core_axis_name="c", subcore_axis_name="s"),
    scratch_types=[
        pltpu.VMEM((NI, 2, CHUNK), jnp.int32),
        pltpu.VMEM((NBUF, CHUNK, D), jnp.float32),
        pltpu.VMEM_SHARED((N_PAD, D), jnp.float32),
        pltpu.SemaphoreType.DMA((NI,)),
        pltpu.SemaphoreType.DMA((NBUF,)),
        pltpu.SemaphoreType.DMA((NBUF,)),
    ],
)


BLK = 400
GRID = N // BLK


def _fused_body(h_ref, a0_ref, a1_ref, w1_ref, b1_ref, w2_ref, b2_ref,
                g_ref, be_ref, o_ref, z_sc, s_sc, ss_sc):
    p = pl.program_id(0)
    i = pl.program_id(1)

    @pl.when(p == 0)
    def _mlp():
        zin = h_ref[...] + a0_ref[0] + a1_ref[0]
        t = jnp.dot(zin, w1_ref[...], preferred_element_type=jnp.float32)
        t = jnp.maximum(t + b1_ref[...], 0.0)
        z2 = jnp.dot(t, w2_ref[...], preferred_element_type=jnp.float32)
        z2 = jnp.maximum(z2 + b2_ref[...], 0.0)
        z_sc[pl.ds(i * BLK, BLK), :] = z2
        ps = jnp.sum(z2.reshape(BLK // 8, 8, D), axis=0)
        pss = jnp.sum((z2 * z2).reshape(BLK // 8, 8, D), axis=0)

        @pl.when(i == 0)
        def _init():
            s_sc[...] = ps
            ss_sc[...] = pss

        @pl.when(i > 0)
        def _acc():
            s_sc[...] += ps
            ss_sc[...] += pss

    @pl.when(p == 1)
    def _bn():
        mean = jnp.sum(s_sc[...], axis=0, keepdims=True) * (1.0 / N)
        msq = jnp.sum(ss_sc[...], axis=0, keepdims=True) * (1.0 / N)
        var = msq - mean * mean
        inv = lax.rsqrt(var + 1e-5)
        scale = g_ref[...] * inv
        shift = be_ref[...] - mean * scale
        o_ref[...] = z_sc[pl.ds(i * BLK, BLK), :] * scale + shift


_fused = pl.pallas_call(
    _fused_body,
    grid=(2, GRID),
    in_specs=[
        pl.BlockSpec((BLK, D), lambda p, i: (i * (1 - p), 0)),
        pl.BlockSpec((1, BLK, D), lambda p, i: (0, i * (1 - p), 0)),
        pl.BlockSpec((1, BLK, D), lambda p, i: (1, i * (1 - p), 0)),
        pl.BlockSpec((D, D), lambda p, i: (0, 0)),
        pl.BlockSpec((1, D), lambda p, i: (0, 0)),
        pl.BlockSpec((D, D), lambda p, i: (0, 0)),
        pl.BlockSpec((1, D), lambda p, i: (0, 0)),
        pl.BlockSpec((1, D), lambda p, i: (0, 0)),
        pl.BlockSpec((1, D), lambda p, i: (0, 0)),
    ],
    out_specs=pl.BlockSpec((BLK, D), lambda p, i: (i * p, 0)),
    out_shape=jax.ShapeDtypeStruct((N, D), jnp.float32),
    scratch_shapes=[
        pltpu.VMEM((N, D), jnp.float32),
        pltpu.VMEM((8, D), jnp.float32),
        pltpu.VMEM((8, D), jnp.float32),
    ],
)


def kernel(x, edge_index,
           W1_0, b1_0, W2_0, b2_0, gamma_0, beta_0,
           W1_1, b1_1, W2_1, b2_1, gamma_1, beta_1,
           W1_2, b1_2, W2_2, b2_2, gamma_2, beta_2):
    params = [
        (W1_0, b1_0, W2_0, b2_0, gamma_0, beta_0),
        (W1_1, b1_1, W2_1, b2_1, gamma_1, beta_1),
        (W1_2, b1_2, W2_2, b2_2, gamma_2, beta_2),
    ]
    npad = E_PAD - E
    pad = jnp.stack([
        jnp.arange(npad, dtype=jnp.int32) % N,
        N + (jnp.arange(npad, dtype=jnp.int32) % (N_PAD - N)),
    ])
    ei = jnp.concatenate([edge_index, pad], axis=1)
    ei = ei.reshape(2, NW, CPW, CHUNK).transpose(1, 2, 0, 3)

    h = x
    for (W1, b1, W2, b2, g, be) in params:
        agg = _sc_agg(h, ei)
        h = _fused(h, agg, agg, W1, b1.reshape(1, D), W2, b2.reshape(1, D),
                   g.reshape(1, D), be.reshape(1, D))
    return h


# speedup vs baseline: 1.2849x; 1.0442x over previous
import functools

import jax
import jax.numpy as jnp
from jax import lax
from jax.experimental import pallas as pl
from jax.experimental.pallas import tpu as pltpu
from jax.experimental.pallas import tpu_sc as plsc

N = 10000
D = 128
E = 320000
L = 3

NC = 2
NS = 16
NW = NC * NS

CHUNK = 64
CPW = 160
EPW = CHUNK * CPW
E_PAD = NW * EPW
N_PAD = 10240
ROWS_PER_TILE = N_PAD // NS
DUMMY_DST = N_PAD - 8
NBUF = 5
GLA = 4
NI = 5


def _sc_agg_body(h_hbm, ei_hbm, out_hbm, idx, rows, agg_sh,
                 sem_i, sem_g, sem_s, sem_o):
    c = lax.axis_index("c")
    s = lax.axis_index("s")
    wid = s * NC + c

    def _start_i(j):
        sl = lax.rem(j, NI)
        pltpu.async_copy(ei_hbm.at[wid, j], idx.at[sl], sem_i.at[sl])

    def _wait_i(j):
        sl = lax.rem(j, NI)
        pltpu.make_async_copy(ei_hbm.at[wid, j], idx.at[sl],
                              sem_i.at[sl]).wait()

    def _start_g(j):
        sl = lax.rem(j, NI)
        b = lax.rem(j, NBUF)
        pltpu.async_copy(h_hbm.at[idx.at[sl, 0]], rows.at[b], sem_g.at[b])

    def _wait_g(j):
        sl = lax.rem(j, NI)
        b = lax.rem(j, NBUF)
        pltpu.make_async_copy(h_hbm.at[idx.at[sl, 0]], rows.at[b],
                              sem_g.at[b]).wait()

    def _start_s(j):
        sl = lax.rem(j, NI)
        b = lax.rem(j, NBUF)
        pltpu.async_copy(rows.at[b], agg_sh.at[idx.at[sl, 1]], sem_s.at[b],
                         add=True)

    def _wait_s(j):
        sl = lax.rem(j, NI)
        b = lax.rem(j, NBUF)
        pltpu.make_async_copy(rows.at[b], agg_sh.at[idx.at[sl, 1]],
                              sem_s.at[b]).wait()

    for j in range(GLA + 1):
        _start_i(j)
    for j in range(GLA):
        _wait_i(j)
        _start_g(j)

    def _zrow(i, _):
        for j in range(D // 16):
            rows[NBUF - 1, i, pl.ds(j * 16, 16)] = jnp.zeros((16,), jnp.float32)
        return _

    lax.fori_loop(0, CHUNK, _zrow, None)
    for t in range(ROWS_PER_TILE // CHUNK):
        r = s * ROWS_PER_TILE + t * CHUNK
        pltpu.async_copy(rows.at[NBUF - 1], agg_sh.at[pl.ds(r, CHUNK)], sem_o)
    for t in range(ROWS_PER_TILE // CHUNK):
        r = s * ROWS_PER_TILE + t * CHUNK
        pltpu.make_async_copy(rows.at[NBUF - 1], agg_sh.at[pl.ds(r, CHUNK)],
                              sem_o).wait()
    plsc.subcore_barrier()

    def _iter(i, _):
        _wait_g(i)
        _start_s(i)
        j2 = i + GLA + 1

        @pl.when(j2 < CPW)
        def _():
            _start_i(j2)
        j = i + GLA

        @pl.when(j < CPW)
        def _():
            @pl.when(j >= NBUF)
            def _():
                _wait_s(j - NBUF)
            _wait_i(j)
            _start_g(j)
        return _

    lax.fori_loop(0, CPW, _iter, None)
    for j in range(CPW - NBUF, CPW):
        _wait_s(j)
    plsc.subcore_barrier()

    for t in range(ROWS_PER_TILE // CHUNK):
        r = s * ROWS_PER_TILE + t * CHUNK
        pltpu.async_copy(agg_sh.at[pl.ds(r, CHUNK)],
                         out_hbm.at[c, pl.ds(r, CHUNK)], sem_o)
    for t in range(ROWS_PER_TILE // CHUNK):
        r = s * ROWS_PER_TILE + t * CHUNK
        pltpu.make_async_copy(agg_sh.at[pl.ds(r, CHUNK)],
                              out_hbm.at[c, pl.ds(r, CHUNK)], sem_o).wait()


_sc_agg = pl.kernel(
    _sc_agg_body,
    out_type=jax.ShapeDtypeStruct((NC, N_PAD, D), jnp.float32),
    mesh=plsc.VectorSubcoreMesh(core_axis_name="c", subcore_axis_name="s"),
    scratch_types=[
        pltpu.VMEM((NI, 2, CHUNK), jnp.int32),
        pltpu.VMEM((NBUF, CHUNK, D), jnp.float32),
        pltpu.VMEM_SHARED((N_PAD, D), jnp.float32),
        pltpu.SemaphoreType.DMA((NI,)),
        pltpu.SemaphoreType.DMA((NBUF,)),
        pltpu.SemaphoreType.DMA((NBUF,)),
        pltpu.SemaphoreType.DMA,
    ],
)


BLK = 400
GRID = N // BLK


def _fused_body(h_ref, a0_ref, a1_ref, w1_ref, b1_ref, w2_ref, b2_ref,
                g_ref, be_ref, o_ref, z_sc, s_sc, ss_sc):
    p = pl.program_id(0)
    i = pl.program_id(1)

    @pl.when(p == 0)
    def _mlp():
        zin = h_ref[...] + a0_ref[0] + a1_ref[0]
        t = jnp.dot(zin, w1_ref[...], preferred_element_type=jnp.float32)
        t = jnp.maximum(t + b1_ref[...], 0.0)
        z2 = jnp.dot(t, w2_ref[...], preferred_element_type=jnp.float32)
        z2 = jnp.maximum(z2 + b2_ref[...], 0.0)
        z_sc[pl.ds(i * BLK, BLK), :] = z2
        ps = jnp.sum(z2.reshape(BLK // 8, 8, D), axis=0)
        pss = jnp.sum((z2 * z2).reshape(BLK // 8, 8, D), axis=0)

        @pl.when(i == 0)
        def _init():
            s_sc[...] = ps
            ss_sc[...] = pss

        @pl.when(i > 0)
        def _acc():
            s_sc[...] += ps
            ss_sc[...] += pss

    @pl.when(p == 1)
    def _bn():
        mean = jnp.sum(s_sc[...], axis=0, keepdims=True) * (1.0 / N)
        msq = jnp.sum(ss_sc[...], axis=0, keepdims=True) * (1.0 / N)
        var = msq - mean * mean
        inv = lax.rsqrt(var + 1e-5)
        scale = g_ref[...] * inv
        shift = be_ref[...] - mean * scale
        o_ref[...] = z_sc[pl.ds(i * BLK, BLK), :] * scale + shift


_fused = pl.pallas_call(
    _fused_body,
    grid=(2, GRID),
    in_specs=[
        pl.BlockSpec((BLK, D), lambda p, i: (i * (1 - p), 0)),
        pl.BlockSpec((1, BLK, D), lambda p, i: (0, i * (1 - p), 0)),
        pl.BlockSpec((1, BLK, D), lambda p, i: (1, i * (1 - p), 0)),
        pl.BlockSpec((D, D), lambda p, i: (0, 0)),
        pl.BlockSpec((1, D), lambda p, i: (0, 0)),
        pl.BlockSpec((D, D), lambda p, i: (0, 0)),
        pl.BlockSpec((1, D), lambda p, i: (0, 0)),
        pl.BlockSpec((1, D), lambda p, i: (0, 0)),
        pl.BlockSpec((1, D), lambda p, i: (0, 0)),
    ],
    out_specs=pl.BlockSpec((BLK, D), lambda p, i: (i * p, 0)),
    out_shape=jax.ShapeDtypeStruct((N, D), jnp.float32),
    scratch_shapes=[
        pltpu.VMEM((N, D), jnp.float32),
        pltpu.VMEM((8, D), jnp.float32),
        pltpu.VMEM((8, D), jnp.float32),
    ],
)


def kernel(x, edge_index,
           W1_0, b1_0, W2_0, b2_0, gamma_0, beta_0,
           W1_1, b1_1, W2_1, b2_1, gamma_1, beta_1,
           W1_2, b1_2, W2_2, b2_2, gamma_2, beta_2):
    params = [
        (W1_0, b1_0, W2_0, b2_0, gamma_0, beta_0),
        (W1_1, b1_1, W2_1, b2_1, gamma_1, beta_1),
        (W1_2, b1_2, W2_2, b2_2, gamma_2, beta_2),
    ]
    npad = E_PAD - E
    pad = jnp.stack([
        jnp.arange(npad, dtype=jnp.int32) % N,
        N + (jnp.arange(npad, dtype=jnp.int32) % (N_PAD - N)),
    ])
    ei = jnp.concatenate([edge_index, pad], axis=1)
    ei = ei.reshape(2, NW, CPW, CHUNK).transpose(1, 2, 0, 3)

    h = x
    for (W1, b1, W2, b2, g, be) in params:
        agg = _sc_agg(h, ei)
        h = _fused(h, agg, agg, W1, b1.reshape(1, D), W2, b2.reshape(1, D),
                   g.reshape(1, D), be.reshape(1, D))
    return h


# speedup vs baseline: 1.2924x; 1.0059x over previous
import functools

import jax
import jax.numpy as jnp
from jax import lax
from jax.experimental import pallas as pl
from jax.experimental.pallas import tpu as pltpu
from jax.experimental.pallas import tpu_sc as plsc

N = 10000
D = 128
E = 320000
L = 3

NC = 2
NS = 16
NW = NC * NS

CHUNK = 64
CPW = 160
EPW = CHUNK * CPW
E_PAD = NW * EPW
N_PAD = 10240
ROWS_PER_TILE = N_PAD // NS
DUMMY_DST = N_PAD - 8
NBUF = 5
GLA = 4
NI = 6


def _sc_agg_body(h_hbm, ei_hbm, out_hbm, idx, rows, agg_sh,
                 sem_i, sem_g, sem_s, sem_o):
    c = lax.axis_index("c")
    s = lax.axis_index("s")
    wid = s * NC + c

    def _start_i(j):
        sl = lax.rem(j, NI)
        pltpu.async_copy(ei_hbm.at[wid, j], idx.at[sl], sem_i.at[sl])

    def _wait_i(j):
        sl = lax.rem(j, NI)
        pltpu.make_async_copy(ei_hbm.at[wid, j], idx.at[sl],
                              sem_i.at[sl]).wait()

    def _start_g(j):
        sl = lax.rem(j, NI)
        b = lax.rem(j, NBUF)
        pltpu.async_copy(h_hbm.at[idx.at[sl, 0]], rows.at[b], sem_g.at[b])

    def _wait_g(j):
        sl = lax.rem(j, NI)
        b = lax.rem(j, NBUF)
        pltpu.make_async_copy(h_hbm.at[idx.at[sl, 0]], rows.at[b],
                              sem_g.at[b]).wait()

    def _start_s(j):
        sl = lax.rem(j, NI)
        b = lax.rem(j, NBUF)
        pltpu.async_copy(rows.at[b], agg_sh.at[idx.at[sl, 1]], sem_s.at[b],
                         add=True)

    def _wait_s(j):
        sl = lax.rem(j, NI)
        b = lax.rem(j, NBUF)
        pltpu.make_async_copy(rows.at[b], agg_sh.at[idx.at[sl, 1]],
                              sem_s.at[b]).wait()

    for j in range(GLA + 1):
        _start_i(j)
    for j in range(GLA):
        _wait_i(j)
        _start_g(j)

    def _zrow(i, _):
        for j in range(D // 16):
            rows[NBUF - 1, i, pl.ds(j * 16, 16)] = jnp.zeros((16,), jnp.float32)
        return _

    lax.fori_loop(0, CHUNK, _zrow, None)
    for t in range(ROWS_PER_TILE // CHUNK):
        r = s * ROWS_PER_TILE + t * CHUNK
        pltpu.async_copy(rows.at[NBUF - 1], agg_sh.at[pl.ds(r, CHUNK)], sem_o)
    for t in range(ROWS_PER_TILE // CHUNK):
        r = s * ROWS_PER_TILE + t * CHUNK
        pltpu.make_async_copy(rows.at[NBUF - 1], agg_sh.at[pl.ds(r, CHUNK)],
                              sem_o).wait()
    plsc.subcore_barrier()

    def _iter(i, _):
        j2 = i + GLA + 1

        @pl.when(j2 < CPW)
        def _():
            _start_i(j2)
        _wait_g(i)
        _start_s(i)
        j = i + GLA

        @pl.when(j < CPW)
        def _():
            @pl.when(j >= NBUF)
            def _():
                _wait_s(j - NBUF)
            _wait_i(j)
            _start_g(j)
        return _

    lax.fori_loop(0, CPW, _iter, None)
    for j in range(CPW - NBUF, CPW):
        _wait_s(j)
    plsc.subcore_barrier()

    for t in range(ROWS_PER_TILE // CHUNK):
        r = s * ROWS_PER_TILE + t * CHUNK
        pltpu.async_copy(agg_sh.at[pl.ds(r, CHUNK)],
                         out_hbm.at[c, pl.ds(r, CHUNK)], sem_o)
    for t in range(ROWS_PER_TILE // CHUNK):
        r = s * ROWS_PER_TILE + t * CHUNK
        pltpu.make_async_copy(agg_sh.at[pl.ds(r, CHUNK)],
                              out_hbm.at[c, pl.ds(r, CHUNK)], sem_o).wait()


_sc_agg = pl.kernel(
    _sc_agg_body,
    out_type=jax.ShapeDtypeStruct((NC, N_PAD, D), jnp.float32),
    mesh=plsc.VectorSubcoreMesh(core_axis_name="c", subcore_axis_name="s"),
    scratch_types=[
        pltpu.VMEM((NI, 2, CHUNK), jnp.int32),
        pltpu.VMEM((NBUF, CHUNK, D), jnp.float32),
        pltpu.VMEM_SHARED((N_PAD, D), jnp.float32),
        pltpu.SemaphoreType.DMA((NI,)),
        pltpu.SemaphoreType.DMA((NBUF,)),
        pltpu.SemaphoreType.DMA((NBUF,)),
        pltpu.SemaphoreType.DMA,
    ],
)


BLK = 400
GRID = N // BLK


def _fused_body(h_ref, a0_ref, a1_ref, w1_ref, b1_ref, w2_ref, b2_ref,
                g_ref, be_ref, o_ref, z_sc, s_sc, ss_sc):
    p = pl.program_id(0)
    i = pl.program_id(1)

    @pl.when(p == 0)
    def _mlp():
        zin = h_ref[...] + a0_ref[0] + a1_ref[0]
        t = jnp.dot(zin, w1_ref[...], preferred_element_type=jnp.float32)
        t = jnp.maximum(t + b1_ref[...], 0.0)
        z2 = jnp.dot(t, w2_ref[...], preferred_element_type=jnp.float32)
        z2 = jnp.maximum(z2 + b2_ref[...], 0.0)
        z_sc[pl.ds(i * BLK, BLK), :] = z2
        ps = jnp.sum(z2.reshape(BLK // 8, 8, D), axis=0)
        pss = jnp.sum((z2 * z2).reshape(BLK // 8, 8, D), axis=0)

        @pl.when(i == 0)
        def _init():
            s_sc[...] = ps
            ss_sc[...] = pss

        @pl.when(i > 0)
        def _acc():
            s_sc[...] += ps
            ss_sc[...] += pss

    @pl.when(p == 1)
    def _bn():
        mean = jnp.sum(s_sc[...], axis=0, keepdims=True) * (1.0 / N)
        msq = jnp.sum(ss_sc[...], axis=0, keepdims=True) * (1.0 / N)
        var = msq - mean * mean
        inv = lax.rsqrt(var + 1e-5)
        scale = g_ref[...] * inv
        shift = be_ref[...] - mean * scale
        o_ref[...] = z_sc[pl.ds(i * BLK, BLK), :] * scale + shift


_fused = pl.pallas_call(
    _fused_body,
    grid=(2, GRID),
    in_specs=[
        pl.BlockSpec((BLK, D), lambda p, i: (i * (1 - p), 0)),
        pl.BlockSpec((1, BLK, D), lambda p, i: (0, i * (1 - p), 0)),
        pl.BlockSpec((1, BLK, D), lambda p, i: (1, i * (1 - p), 0)),
        pl.BlockSpec((D, D), lambda p, i: (0, 0)),
        pl.BlockSpec((1, D), lambda p, i: (0, 0)),
        pl.BlockSpec((D, D), lambda p, i: (0, 0)),
        pl.BlockSpec((1, D), lambda p, i: (0, 0)),
        pl.BlockSpec((1, D), lambda p, i: (0, 0)),
        pl.BlockSpec((1, D), lambda p, i: (0, 0)),
    ],
    out_specs=pl.BlockSpec((BLK, D), lambda p, i: (i * p, 0)),
    out_shape=jax.ShapeDtypeStruct((N, D), jnp.float32),
    scratch_shapes=[
        pltpu.VMEM((N, D), jnp.float32),
        pltpu.VMEM((8, D), jnp.float32),
        pltpu.VMEM((8, D), jnp.float32),
    ],
)


def kernel(x, edge_index,
           W1_0, b1_0, W2_0, b2_0, gamma_0, beta_0,
           W1_1, b1_1, W2_1, b2_1, gamma_1, beta_1,
           W1_2, b1_2, W2_2, b2_2, gamma_2, beta_2):
    params = [
        (W1_0, b1_0, W2_0, b2_0, gamma_0, beta_0),
        (W1_1, b1_1, W2_1, b2_1, gamma_1, beta_1),
        (W1_2, b1_2, W2_2, b2_2, gamma_2, beta_2),
    ]
    npad = E_PAD - E
    pad = jnp.stack([
        jnp.arange(npad, dtype=jnp.int32) % N,
        N + (jnp.arange(npad, dtype=jnp.int32) % (N_PAD - N)),
    ])
    ei = jnp.concatenate([edge_index, pad], axis=1)
    ei = ei.reshape(2, NW, CPW, CHUNK).transpose(1, 2, 0, 3)

    h = x
    for (W1, b1, W2, b2, g, be) in params:
        agg = _sc_agg(h, ei)
        h = _fused(h, agg, agg, W1, b1.reshape(1, D), W2, b2.reshape(1, D),
                   g.reshape(1, D), be.reshape(1, D))
    return h


# speedup vs baseline: 1.2934x; 1.0008x over previous
import functools

import jax
import jax.numpy as jnp
from jax import lax
from jax.experimental import pallas as pl
from jax.experimental.pallas import tpu as pltpu
from jax.experimental.pallas import tpu_sc as plsc

N = 10000
D = 128
E = 320000
L = 3

NC = 2
NS = 16
NW = NC * NS

CHUNK = 64
CPW = 160
EPW = CHUNK * CPW
E_PAD = NW * EPW
N_PAD = 10240
ROWS_PER_TILE = N_PAD // NS
DUMMY_DST = N_PAD - 8
NBUF = 5
GLA = 4
NI = 7


def _sc_agg_body(h_hbm, ei_hbm, out_hbm, idx, rows, agg_sh,
                 sem_i, sem_g, sem_s, sem_o):
    c = lax.axis_index("c")
    s = lax.axis_index("s")
    wid = s * NC + c

    def _start_i(j):
        sl = lax.rem(j, NI)
        pltpu.async_copy(ei_hbm.at[wid, j], idx.at[sl], sem_i.at[sl])

    def _wait_i(j):
        sl = lax.rem(j, NI)
        pltpu.make_async_copy(ei_hbm.at[wid, j], idx.at[sl],
                              sem_i.at[sl]).wait()

    def _start_g(j):
        sl = lax.rem(j, NI)
        b = lax.rem(j, NBUF)
        pltpu.async_copy(h_hbm.at[idx.at[sl, 0]], rows.at[b], sem_g.at[b])

    def _wait_g(j):
        sl = lax.rem(j, NI)
        b = lax.rem(j, NBUF)
        pltpu.make_async_copy(h_hbm.at[idx.at[sl, 0]], rows.at[b],
                              sem_g.at[b]).wait()

    def _start_s(j):
        sl = lax.rem(j, NI)
        b = lax.rem(j, NBUF)
        pltpu.async_copy(rows.at[b], agg_sh.at[idx.at[sl, 1]], sem_s.at[b],
                         add=True)

    def _wait_s(j):
        sl = lax.rem(j, NI)
        b = lax.rem(j, NBUF)
        pltpu.make_async_copy(rows.at[b], agg_sh.at[idx.at[sl, 1]],
                              sem_s.at[b]).wait()

    for j in range(GLA + 1):
        _start_i(j)
    for j in range(GLA):
        _wait_i(j)
        _start_g(j)

    def _zrow(i, _):
        for j in range(D // 16):
            rows[NBUF - 1, i, pl.ds(j * 16, 16)] = jnp.zeros((16,), jnp.float32)
        return _

    lax.fori_loop(0, CHUNK, _zrow, None)
    for t in range(ROWS_PER_TILE // CHUNK):
        r = s * ROWS_PER_TILE + t * CHUNK
        pltpu.async_copy(rows.at[NBUF - 1], agg_sh.at[pl.ds(r, CHUNK)], sem_o)
    for t in range(ROWS_PER_TILE // CHUNK):
        r = s * ROWS_PER_TILE + t * CHUNK
        pltpu.make_async_copy(rows.at[NBUF - 1], agg_sh.at[pl.ds(r, CHUNK)],
                              sem_o).wait()
    plsc.subcore_barrier()

    def _iter(i, _):
        j2 = i + GLA + 1

        @pl.when(j2 < CPW)
        def _():
            _start_i(j2)
        _wait_g(i)
        _start_s(i)
        j = i + GLA

        @pl.when(j < CPW)
        def _():
            @pl.when(j >= NBUF)
            def _():
                _wait_s(j - NBUF)
            _wait_i(j)
            _start_g(j)
        return _

    lax.fori_loop(0, CPW, _iter, None)
    for j in range(CPW - NBUF, CPW):
        _wait_s(j)
    plsc.subcore_barrier()

    for t in range(ROWS_PER_TILE // CHUNK):
        r = s * ROWS_PER_TILE + t * CHUNK
        pltpu.async_copy(agg_sh.at[pl.ds(r, CHUNK)],
                         out_hbm.at[c, pl.ds(r, CHUNK)], sem_o)
    for t in range(ROWS_PER_TILE // CHUNK):
        r = s * ROWS_PER_TILE + t * CHUNK
        pltpu.make_async_copy(agg_sh.at[pl.ds(r, CHUNK)],
                              out_hbm.at[c, pl.ds(r, CHUNK)], sem_o).wait()


_sc_agg = pl.kernel(
    _sc_agg_body,
    out_type=jax.ShapeDtypeStruct((NC, N_PAD, D), jnp.float32),
    mesh=plsc.VectorSubcoreMesh(core_axis_name="c", subcore_axis_name="s"),
    scratch_types=[
        pltpu.VMEM((NI, 2, CHUNK), jnp.int32),
        pltpu.VMEM((NBUF, CHUNK, D), jnp.float32),
        pltpu.VMEM_SHARED((N_PAD, D), jnp.float32),
        pltpu.SemaphoreType.DMA((NI,)),
        pltpu.SemaphoreType.DMA((NBUF,)),
        pltpu.SemaphoreType.DMA((NBUF,)),
        pltpu.SemaphoreType.DMA,
    ],
)


BLK = 400
GRID = N // BLK


def _fused_body(h_ref, a0_ref, a1_ref, w1_ref, b1_ref, w2_ref, b2_ref,
                g_ref, be_ref, o_ref, z_sc, s_sc, ss_sc):
    p = pl.program_id(0)
    i = pl.program_id(1)

    @pl.when(p == 0)
    def _mlp():
        zin = h_ref[...] + a0_ref[0] + a1_ref[0]
        t = jnp.dot(zin, w1_ref[...], preferred_element_type=jnp.float32)
        t = jnp.maximum(t + b1_ref[...], 0.0)
        z2 = jnp.dot(t, w2_ref[...], preferred_element_type=jnp.float32)
        z2 = jnp.maximum(z2 + b2_ref[...], 0.0)
        z_sc[pl.ds(i * BLK, BLK), :] = z2
        ps = jnp.sum(z2.reshape(BLK // 8, 8, D), axis=0)
        pss = jnp.sum((z2 * z2).reshape(BLK // 8, 8, D), axis=0)

        @pl.when(i == 0)
        def _init():
            s_sc[...] = ps
            ss_sc[...] = pss

        @pl.when(i > 0)
        def _acc():
            s_sc[...] += ps
            ss_sc[...] += pss

    @pl.when(p == 1)
    def _bn():
        mean = jnp.sum(s_sc[...], axis=0, keepdims=True) * (1.0 / N)
        msq = jnp.sum(ss_sc[...], axis=0, keepdims=True) * (1.0 / N)
        var = msq - mean * mean
        inv = lax.rsqrt(var + 1e-5)
        scale = g_ref[...] * inv
        shift = be_ref[...] - mean * scale
        o_ref[...] = z_sc[pl.ds(i * BLK, BLK), :] * scale + shift


_fused = pl.pallas_call(
    _fused_body,
    grid=(2, GRID),
    in_specs=[
        pl.BlockSpec((BLK, D), lambda p, i: (i * (1 - p), 0)),
        pl.BlockSpec((1, BLK, D), lambda p, i: (0, i * (1 - p), 0)),
        pl.BlockSpec((1, BLK, D), lambda p, i: (1, i * (1 - p), 0)),
        pl.BlockSpec((D, D), lambda p, i: (0, 0)),
        pl.BlockSpec((1, D), lambda p, i: (0, 0)),
        pl.BlockSpec((D, D), lambda p, i: (0, 0)),
        pl.BlockSpec((1, D), lambda p, i: (0, 0)),
        pl.BlockSpec((1, D), lambda p, i: (0, 0)),
        pl.BlockSpec((1, D), lambda p, i: (0, 0)),
    ],
    out_specs=pl.BlockSpec((BLK, D), lambda p, i: (i * p, 0)),
    out_shape=jax.ShapeDtypeStruct((N, D), jnp.float32),
    scratch_shapes=[
        pltpu.VMEM((N, D), jnp.float32),
        pltpu.VMEM((8, D), jnp.float32),
        pltpu.VMEM((8, D), jnp.float32),
    ],
)


def kernel(x, edge_index,
           W1_0, b1_0, W2_0, b2_0, gamma_0, beta_0,
           W1_1, b1_1, W2_1, b2_1, gamma_1, beta_1,
           W1_2, b1_2, W2_2, b2_2, gamma_2, beta_2):
    params = [
        (W1_0, b1_0, W2_0, b2_0, gamma_0, beta_0),
        (W1_1, b1_1, W2_1, b2_1, gamma_1, beta_1),
        (W1_2, b1_2, W2_2, b2_2, gamma_2, beta_2),
    ]
    npad = E_PAD - E
    pad = jnp.stack([
        jnp.arange(npad, dtype=jnp.int32) % N,
        N + (jnp.arange(npad, dtype=jnp.int32) % (N_PAD - N)),
    ])
    ei = jnp.concatenate([edge_index, pad], axis=1)
    ei = ei.reshape(2, NW, CPW, CHUNK).transpose(1, 2, 0, 3)

    h = x
    for (W1, b1, W2, b2, g, be) in params:
        agg = _sc_agg(h, ei)
        h = _fused(h, agg, agg, W1, b1.reshape(1, D), W2, b2.reshape(1, D),
                   g.reshape(1, D), be.reshape(1, D))
    return h


# speedup vs baseline: 1.4324x; 1.1074x over previous
import functools

import jax
import jax.numpy as jnp
from jax import lax
from jax.experimental import pallas as pl
from jax.experimental.pallas import tpu as pltpu
from jax.experimental.pallas import tpu_sc as plsc

N = 10000
D = 128
E = 320000
L = 3

NC = 2
NS = 16
NW = NC * NS

CHUNK = 64
CPW = 160
EPW = CHUNK * CPW
E_PAD = NW * EPW
N_PAD = 10240
ROWS_PER_TILE = N_PAD // NS
DUMMY_DST = N_PAD - 8
NBUF = 5
GLA = 4
NI = 7


def _sc_agg_body(h_hbm, ei_hbm, out_hbm, idx, rows, agg_sh,
                 sem_i, sem_g, sem_s, sem_o):
    c = lax.axis_index("c")
    s = lax.axis_index("s")
    wid = s * NC + c

    def _start_i(j):
        sl = lax.rem(j, NI)
        pltpu.async_copy(ei_hbm.at[wid, j], idx.at[sl], sem_i.at[sl])

    def _wait_i(j):
        sl = lax.rem(j, NI)
        pltpu.make_async_copy(ei_hbm.at[wid, j], idx.at[sl],
                              sem_i.at[sl]).wait()

    def _start_g(j):
        sl = lax.rem(j, NI)
        b = lax.rem(j, NBUF)
        pltpu.async_copy(h_hbm.at[idx.at[sl, 0]], rows.at[b], sem_g.at[b])

    def _wait_g(j):
        sl = lax.rem(j, NI)
        b = lax.rem(j, NBUF)
        pltpu.make_async_copy(h_hbm.at[idx.at[sl, 0]], rows.at[b],
                              sem_g.at[b]).wait()

    def _start_s(j):
        sl = lax.rem(j, NI)
        b = lax.rem(j, NBUF)
        pltpu.async_copy(rows.at[b], agg_sh.at[idx.at[sl, 1]], sem_s.at[b],
                         add=True)

    def _wait_s(j):
        sl = lax.rem(j, NI)
        b = lax.rem(j, NBUF)
        pltpu.make_async_copy(rows.at[b], agg_sh.at[idx.at[sl, 1]],
                              sem_s.at[b]).wait()

    for j in range(GLA + 1):
        _start_i(j)
    for j in range(GLA):
        _wait_i(j)
        _start_g(j)

    def _zrow(i, _):
        for j in range(D // 16):
            rows[NBUF - 1, i, pl.ds(j * 16, 16)] = jnp.zeros((16,), jnp.float32)
        return _

    lax.fori_loop(0, CHUNK, _zrow, None)
    for t in range(ROWS_PER_TILE // CHUNK):
        r = s * ROWS_PER_TILE + t * CHUNK
        pltpu.async_copy(rows.at[NBUF - 1], agg_sh.at[pl.ds(r, CHUNK)], sem_o)
    for t in range(ROWS_PER_TILE // CHUNK):
        r = s * ROWS_PER_TILE + t * CHUNK
        pltpu.make_async_copy(rows.at[NBUF - 1], agg_sh.at[pl.ds(r, CHUNK)],
                              sem_o).wait()
    plsc.subcore_barrier()

    def _iter(i, _):
        j2 = i + GLA + 1

        @pl.when(j2 < CPW)
        def _():
            _start_i(j2)
        _wait_g(i)
        _start_s(i)
        j = i + GLA

        @pl.when(j < CPW)
        def _():
            @pl.when(j >= NBUF)
            def _():
                _wait_s(j - NBUF)
            _wait_i(j)
            _start_g(j)
        return _

    lax.fori_loop(0, CPW, _iter, None)
    for j in range(CPW - NBUF, CPW):
        _wait_s(j)
    plsc.subcore_barrier()

    for t in range(ROWS_PER_TILE // CHUNK):
        r = s * ROWS_PER_TILE + t * CHUNK
        pltpu.async_copy(agg_sh.at[pl.ds(r, CHUNK)],
                         out_hbm.at[c, pl.ds(r, CHUNK)], sem_o)
    for t in range(ROWS_PER_TILE // CHUNK):
        r = s * ROWS_PER_TILE + t * CHUNK
        pltpu.make_async_copy(agg_sh.at[pl.ds(r, CHUNK)],
                              out_hbm.at[c, pl.ds(r, CHUNK)], sem_o).wait()


_sc_agg = pl.kernel(
    _sc_agg_body,
    out_type=jax.ShapeDtypeStruct((NC, N_PAD, D), jnp.float32),
    mesh=plsc.VectorSubcoreMesh(core_axis_name="c", subcore_axis_name="s"),
    scratch_types=[
        pltpu.VMEM((NI, 2, CHUNK), jnp.int32),
        pltpu.VMEM((NBUF, CHUNK, D), jnp.float32),
        pltpu.VMEM_SHARED((N_PAD, D), jnp.float32),
        pltpu.SemaphoreType.DMA((NI,)),
        pltpu.SemaphoreType.DMA((NBUF,)),
        pltpu.SemaphoreType.DMA((NBUF,)),
        pltpu.SemaphoreType.DMA,
    ],
)


BLK = 1000
GRID = N // BLK


def _fused_body(h_ref, a0_ref, a1_ref, w1_ref, b1_ref, w2_ref, b2_ref,
                g_ref, be_ref, o_ref, z_sc, s_sc, ss_sc):
    p = pl.program_id(0)
    i = pl.program_id(1)

    @pl.when(p == 0)
    def _mlp():
        zin = h_ref[...] + a0_ref[0] + a1_ref[0]
        t = jnp.dot(zin, w1_ref[...], preferred_element_type=jnp.float32)
        t = jnp.maximum(t + b1_ref[...], 0.0)
        z2 = jnp.dot(t, w2_ref[...], preferred_element_type=jnp.float32)
        z2 = jnp.maximum(z2 + b2_ref[...], 0.0)
        z_sc[pl.ds(i * BLK, BLK), :] = z2
        ps = jnp.sum(z2.reshape(BLK // 8, 8, D), axis=0)
        pss = jnp.sum((z2 * z2).reshape(BLK // 8, 8, D), axis=0)

        @pl.when(i == 0)
        def _init():
            s_sc[...] = ps
            ss_sc[...] = pss

        @pl.when(i > 0)
        def _acc():
            s_sc[...] += ps
            ss_sc[...] += pss

    @pl.when(p == 1)
    def _bn():
        mean = jnp.sum(s_sc[...], axis=0, keepdims=True) * (1.0 / N)
        msq = jnp.sum(ss_sc[...], axis=0, keepdims=True) * (1.0 / N)
        var = msq - mean * mean
        inv = lax.rsqrt(var + 1e-5)
        scale = g_ref[...] * inv
        shift = be_ref[...] - mean * scale
        o_ref[...] = z_sc[pl.ds(i * BLK, BLK), :] * scale + shift


_fused = pl.pallas_call(
    _fused_body,
    grid=(2, GRID),
    in_specs=[
        pl.BlockSpec((BLK, D), lambda p, i: (i * (1 - p), 0)),
        pl.BlockSpec((1, BLK, D), lambda p, i: (0, i * (1 - p), 0)),
        pl.BlockSpec((1, BLK, D), lambda p, i: (1, i * (1 - p), 0)),
        pl.BlockSpec((D, D), lambda p, i: (0, 0)),
        pl.BlockSpec((1, D), lambda p, i: (0, 0)),
        pl.BlockSpec((D, D), lambda p, i: (0, 0)),
        pl.BlockSpec((1, D), lambda p, i: (0, 0)),
        pl.BlockSpec((1, D), lambda p, i: (0, 0)),
        pl.BlockSpec((1, D), lambda p, i: (0, 0)),
    ],
    out_specs=pl.BlockSpec((BLK, D), lambda p, i: (i * p, 0)),
    out_shape=jax.ShapeDtypeStruct((N, D), jnp.float32),
    scratch_shapes=[
        pltpu.VMEM((N, D), jnp.float32),
        pltpu.VMEM((8, D), jnp.float32),
        pltpu.VMEM((8, D), jnp.float32),
    ],
)


def kernel(x, edge_index,
           W1_0, b1_0, W2_0, b2_0, gamma_0, beta_0,
           W1_1, b1_1, W2_1, b2_1, gamma_1, beta_1,
           W1_2, b1_2, W2_2, b2_2, gamma_2, beta_2):
    params = [
        (W1_0, b1_0, W2_0, b2_0, gamma_0, beta_0),
        (W1_1, b1_1, W2_1, b2_1, gamma_1, beta_1),
        (W1_2, b1_2, W2_2, b2_2, gamma_2, beta_2),
    ]
    npad = E_PAD - E
    pad = jnp.stack([
        jnp.arange(npad, dtype=jnp.int32) % N,
        N + (jnp.arange(npad, dtype=jnp.int32) % (N_PAD - N)),
    ])
    ei = jnp.concatenate([edge_index, pad], axis=1)
    ei = ei.reshape(2, NW, CPW, CHUNK).transpose(1, 2, 0, 3)

    h = x
    for (W1, b1, W2, b2, g, be) in params:
        agg = _sc_agg(h, ei)
        h = _fused(h, agg, agg, W1, b1.reshape(1, D), W2, b2.reshape(1, D),
                   g.reshape(1, D), be.reshape(1, D))
    return h


# speedup vs baseline: 1.4910x; 1.0409x over previous
import functools

import jax
import jax.numpy as jnp
from jax import lax
from jax.experimental import pallas as pl
from jax.experimental.pallas import tpu as pltpu
from jax.experimental.pallas import tpu_sc as plsc

N = 10000
D = 128
E = 320000
L = 3

NC = 2
NS = 16
NW = NC * NS

CHUNK = 64
CPW = 160
EPW = CHUNK * CPW
E_PAD = NW * EPW
N_PAD = 10240
ROWS_PER_TILE = N_PAD // NS
DUMMY_DST = N_PAD - 8
NBUF = 5
GLA = 4
NI = 7


def _sc_agg_body(h_hbm, ei_hbm, out_hbm, idx, rows, agg_sh,
                 sem_i, sem_g, sem_s, sem_o):
    c = lax.axis_index("c")
    s = lax.axis_index("s")
    wid = s * NC + c

    def _start_i(j):
        sl = lax.rem(j, NI)
        pltpu.async_copy(ei_hbm.at[wid, j], idx.at[sl], sem_i.at[sl])

    def _wait_i(j):
        sl = lax.rem(j, NI)
        pltpu.make_async_copy(ei_hbm.at[wid, j], idx.at[sl],
                              sem_i.at[sl]).wait()

    def _start_g(j):
        sl = lax.rem(j, NI)
        b = lax.rem(j, NBUF)
        pltpu.async_copy(h_hbm.at[idx.at[sl, 0]], rows.at[b], sem_g.at[b])

    def _wait_g(j):
        sl = lax.rem(j, NI)
        b = lax.rem(j, NBUF)
        pltpu.make_async_copy(h_hbm.at[idx.at[sl, 0]], rows.at[b],
                              sem_g.at[b]).wait()

    def _start_s(j):
        sl = lax.rem(j, NI)
        b = lax.rem(j, NBUF)
        pltpu.async_copy(rows.at[b], agg_sh.at[idx.at[sl, 1]], sem_s.at[b],
                         add=True)

    def _wait_s(j):
        sl = lax.rem(j, NI)
        b = lax.rem(j, NBUF)
        pltpu.make_async_copy(rows.at[b], agg_sh.at[idx.at[sl, 1]],
                              sem_s.at[b]).wait()

    for j in range(GLA + 1):
        _start_i(j)
    for j in range(GLA):
        _wait_i(j)
        _start_g(j)

    def _zrow(i, _):
        for j in range(D // 16):
            rows[NBUF - 1, i, pl.ds(j * 16, 16)] = jnp.zeros((16,), jnp.float32)
        return _

    lax.fori_loop(0, CHUNK, _zrow, None)
    for t in range(ROWS_PER_TILE // CHUNK):
        r = s * ROWS_PER_TILE + t * CHUNK
        pltpu.async_copy(rows.at[NBUF - 1], agg_sh.at[pl.ds(r, CHUNK)], sem_o)
    for t in range(ROWS_PER_TILE // CHUNK):
        r = s * ROWS_PER_TILE + t * CHUNK
        pltpu.make_async_copy(rows.at[NBUF - 1], agg_sh.at[pl.ds(r, CHUNK)],
                              sem_o).wait()
    plsc.subcore_barrier()

    def _iter(i, _):
        j2 = i + GLA + 1

        @pl.when(j2 < CPW)
        def _():
            _start_i(j2)
        _wait_g(i)
        _start_s(i)
        j = i + GLA

        @pl.when(j < CPW)
        def _():
            @pl.when(j >= NBUF)
            def _():
                _wait_s(j - NBUF)
            _wait_i(j)
            _start_g(j)
        return _

    lax.fori_loop(0, CPW, _iter, None)
    for j in range(CPW - NBUF, CPW):
        _wait_s(j)
    plsc.subcore_barrier()

    for t in range(ROWS_PER_TILE // CHUNK):
        r = s * ROWS_PER_TILE + t * CHUNK
        pltpu.async_copy(agg_sh.at[pl.ds(r, CHUNK)],
                         out_hbm.at[c, pl.ds(r, CHUNK)], sem_o)
    for t in range(ROWS_PER_TILE // CHUNK):
        r = s * ROWS_PER_TILE + t * CHUNK
        pltpu.make_async_copy(agg_sh.at[pl.ds(r, CHUNK)],
                              out_hbm.at[c, pl.ds(r, CHUNK)], sem_o).wait()


_sc_agg = pl.kernel(
    _sc_agg_body,
    out_type=jax.ShapeDtypeStruct((NC, N_PAD, D), jnp.float32),
    mesh=plsc.VectorSubcoreMesh(core_axis_name="c", subcore_axis_name="s"),
    scratch_types=[
        pltpu.VMEM((NI, 2, CHUNK), jnp.int32),
        pltpu.VMEM((NBUF, CHUNK, D), jnp.float32),
        pltpu.VMEM_SHARED((N_PAD, D), jnp.float32),
        pltpu.SemaphoreType.DMA((NI,)),
        pltpu.SemaphoreType.DMA((NBUF,)),
        pltpu.SemaphoreType.DMA((NBUF,)),
        pltpu.SemaphoreType.DMA,
    ],
)


BLK = 2000
GRID = N // BLK


def _fused_body(h_ref, a0_ref, a1_ref, w1_ref, b1_ref, w2_ref, b2_ref,
                g_ref, be_ref, o_ref, z_sc, s_sc, ss_sc):
    p = pl.program_id(0)
    i = pl.program_id(1)

    @pl.when(p == 0)
    def _mlp():
        zin = h_ref[...] + a0_ref[0] + a1_ref[0]
        t = jnp.dot(zin, w1_ref[...], preferred_element_type=jnp.float32)
        t = jnp.maximum(t + b1_ref[...], 0.0)
        z2 = jnp.dot(t, w2_ref[...], preferred_element_type=jnp.float32)
        z2 = jnp.maximum(z2 + b2_ref[...], 0.0)
        z_sc[pl.ds(i * BLK, BLK), :] = z2
        ps = jnp.sum(z2.reshape(BLK // 8, 8, D), axis=0)
        pss = jnp.sum((z2 * z2).reshape(BLK // 8, 8, D), axis=0)

        @pl.when(i == 0)
        def _init():
            s_sc[...] = ps
            ss_sc[...] = pss

        @pl.when(i > 0)
        def _acc():
            s_sc[...] += ps
            ss_sc[...] += pss

    @pl.when(p == 1)
    def _bn():
        mean = jnp.sum(s_sc[...], axis=0, keepdims=True) * (1.0 / N)
        msq = jnp.sum(ss_sc[...], axis=0, keepdims=True) * (1.0 / N)
        var = msq - mean * mean
        inv = lax.rsqrt(var + 1e-5)
        scale = g_ref[...] * inv
        shift = be_ref[...] - mean * scale
        o_ref[...] = z_sc[pl.ds(i * BLK, BLK), :] * scale + shift


_fused = pl.pallas_call(
    _fused_body,
    grid=(2, GRID),
    in_specs=[
        pl.BlockSpec((BLK, D), lambda p, i: (i * (1 - p), 0)),
        pl.BlockSpec((1, BLK, D), lambda p, i: (0, i * (1 - p), 0)),
        pl.BlockSpec((1, BLK, D), lambda p, i: (1, i * (1 - p), 0)),
        pl.BlockSpec((D, D), lambda p, i: (0, 0)),
        pl.BlockSpec((1, D), lambda p, i: (0, 0)),
        pl.BlockSpec((D, D), lambda p, i: (0, 0)),
        pl.BlockSpec((1, D), lambda p, i: (0, 0)),
        pl.BlockSpec((1, D), lambda p, i: (0, 0)),
        pl.BlockSpec((1, D), lambda p, i: (0, 0)),
    ],
    out_specs=pl.BlockSpec((BLK, D), lambda p, i: (i * p, 0)),
    out_shape=jax.ShapeDtypeStruct((N, D), jnp.float32),
    scratch_shapes=[
        pltpu.VMEM((N, D), jnp.float32),
        pltpu.VMEM((8, D), jnp.float32),
        pltpu.VMEM((8, D), jnp.float32),
    ],
)


def kernel(x, edge_index,
           W1_0, b1_0, W2_0, b2_0, gamma_0, beta_0,
           W1_1, b1_1, W2_1, b2_1, gamma_1, beta_1,
           W1_2, b1_2, W2_2, b2_2, gamma_2, beta_2):
    params = [
        (W1_0, b1_0, W2_0, b2_0, gamma_0, beta_0),
        (W1_1, b1_1, W2_1, b2_1, gamma_1, beta_1),
        (W1_2, b1_2, W2_2, b2_2, gamma_2, beta_2),
    ]
    npad = E_PAD - E
    pad = jnp.stack([
        jnp.arange(npad, dtype=jnp.int32) % N,
        N + (jnp.arange(npad, dtype=jnp.int32) % (N_PAD - N)),
    ])
    ei = jnp.concatenate([edge_index, pad], axis=1)
    ei = ei.reshape(2, NW, CPW, CHUNK).transpose(1, 2, 0, 3)

    h = x
    for (W1, b1, W2, b2, g, be) in params:
        agg = _sc_agg(h, ei)
        h = _fused(h, agg, agg, W1, b1.reshape(1, D), W2, b2.reshape(1, D),
                   g.reshape(1, D), be.reshape(1, D))
    return h
